# baseline (speedup 1.0000x reference)
import functools

import jax
import jax.numpy as jnp
from jax import lax
from jax.experimental import pallas as pl
from jax.experimental.pallas import tpu as pltpu
from jax.experimental.pallas import tpu_sc as plsc

B, T, D = 4, 256, 128
NPAD = 2568
NROWS = B * T * T
NC, NS = 2, 16
NW = NC * NS
ROWS_PER_W = NROWS // NW
CHUNK = 128
SUP = 256
GPB = SUP // CHUNK
NSUP = ROWS_PER_W // SUP
NBUF = 3
LOOPED = 30


def _tc_body(t_ref, tT_ref, x_ref, W_ref, b_ref, p_ref, ytab_ref, gidx_ref):
    bi = pl.program_id(0)
    delta = p_ref[0, 0, 0]
    tt0 = p_ref[0, 0, 1]
    ext = p_ref[0, 0, 2]

    tr = t_ref[0]
    s = lax.broadcasted_iota(jnp.int32, (NPAD, 1), 0).astype(jnp.float32) * delta

    cmp = (tr <= s).astype(jnp.int32)
    cnt = jnp.sum(cmp, axis=1, keepdims=True)
    cp = jnp.clip(cnt, 1, T - 1)
    kk = lax.broadcasted_iota(jnp.int32, (NPAD, T), 1)
    m1 = (kk == cp - 1).astype(jnp.float32)
    m2 = (kk == cp).astype(jnp.float32)
    t_prev = jnp.sum(m1 * tr, axis=1, keepdims=True)
    t_next = jnp.sum(m2 * tr, axis=1, keepdims=True)
    u = (s - t_prev) / (t_next - t_prev)
    u = jnp.minimum(u, 1.0)
    keep_neg = jnp.logical_and(ext > 0.0, s < tt0)
    u = jnp.where(keep_neg, u, jnp.maximum(u, 0.0))

    A = m1 + u * (m2 - m1)
    xW = lax.dot_general(x_ref[0], W_ref[...],
                         (((1,), (1,)), ((), ())),
                         preferred_element_type=jnp.float32)
    y = lax.dot_general(A, xW, (((1,), (0,)), ((), ())),
                        preferred_element_type=jnp.float32) + b_ref[0:1, :]
    ytab_ref[0] = y

    tcol = tT_ref[0]
    it = tr + (tcol - tr) * jnp.float32(0.5)
    disc = (it / jnp.float32(0.1)).astype(jnp.int32)
    gidx_ref[0] = bi * NPAD + disc


def _make_table_and_idx(x, t, W, b, params):
    t3 = t.reshape(B, 1, T)
    tT = t.reshape(B, T, 1)
    b2 = b.reshape(1, D)
    p3 = params.reshape(B, 1, 8)
    return pl.pallas_call(
        _tc_body,
        grid=(B,),
        in_specs=[
            pl.BlockSpec((1, 1, T), lambda i: (i, 0, 0)),
            pl.BlockSpec((1, T, 1), lambda i: (i, 0, 0)),
            pl.BlockSpec((1, T, D), lambda i: (i, 0, 0)),
            pl.BlockSpec((D, D), lambda i: (0, 0)),
            pl.BlockSpec((1, D), lambda i: (0, 0)),
            pl.BlockSpec((1, 1, 8), lambda i: (i, 0, 0)),
        ],
        out_specs=[
            pl.BlockSpec((1, NPAD, D), lambda i: (i, 0, 0)),
            pl.BlockSpec((1, T, T), lambda i: (i, 0, 0)),
        ],
        out_shape=[
            jax.ShapeDtypeStruct((B, NPAD, D), jnp.float32),
            jax.ShapeDtypeStruct((B, T, T), jnp.int32),
        ],
    )(t3, tT, x, W, b2, p3)


@functools.partial(
    pl.kernel,
    mesh=plsc.VectorSubcoreMesh(core_axis_name="c", subcore_axis_name="s"),
    out_type=jax.ShapeDtypeStruct((NROWS, D), jnp.float32),
    scratch_types=[
        pltpu.VMEM((ROWS_PER_W,), jnp.int32),
    ] + [pltpu.VMEM((SUP, D), jnp.float32)] * NBUF
      + [pltpu.SemaphoreType.DMA] * (2 * NBUF),
)
def _sc_gather(ytab_hbm, gidx_hbm, out_hbm, idx_all, *bufsems):
    rows = bufsems[:NBUF]
    gsem = bufsems[NBUF:2 * NBUF]
    wsem = bufsems[2 * NBUF:]
    wid = lax.axis_index("s") * NC + lax.axis_index("c")
    wbase = wid * ROWS_PER_W

    pltpu.sync_copy(gidx_hbm.at[pl.ds(wbase, ROWS_PER_W)], idx_all)

    def fire_gathers(s, kb):
        for h in range(GPB):
            pltpu.async_copy(
                ytab_hbm.at[idx_all.at[pl.ds(s * SUP + h * CHUNK, CHUNK)]],
                rows[kb].at[pl.ds(h * CHUNK, CHUNK)],
                gsem[kb])

    def drain(sem, kb):
        pltpu.make_async_copy(ytab_hbm.at[pl.ds(0, SUP)], rows[kb], sem).wait()

    def fire_wb(s, kb):
        pltpu.async_copy(rows[kb],
                         out_hbm.at[pl.ds(wbase + s * SUP, SUP)],
                         wsem[kb])

    def step(s, k):
        drain(gsem[k], k)
        kn = (k + NBUF - 1) % NBUF

        @pl.when(s + NBUF - 1 < NSUP)
        def _fire_next():
            @pl.when(s >= 1)
            def _wait_buf():
                drain(wsem[kn], kn)
            fire_gathers(s + NBUF - 1, kn)

        fire_wb(s, k)

    for k in range(NBUF - 1):
        fire_gathers(k, k)

    def body(o, carry):
        for k in range(NBUF):
            step(o * NBUF + k, k)
        return carry

    lax.fori_loop(0, LOOPED // NBUF, body, 0)
    for s in range(LOOPED, NSUP):
        step(s, s % NBUF)
    for k in range(NBUF):
        drain(wsem[k], k)


def kernel(x, t, W, b):
    t = t.astype(jnp.float32)
    x = x.astype(jnp.float32)
    tmax = jnp.max(t)
    a8 = 8.0 * tmax
    a2 = 2.0 * tmax
    f8 = jnp.floor(a8)
    f2 = jnp.floor(a2)
    N = (f8 + f2 + jnp.floor((a8 - f8) + (a2 - f2))).astype(jnp.int32) + 6
    delta = (tmax + 5 * 0.1) / (N - 1).astype(jnp.float32)
    tt0 = jnp.min(t)
    ext = (t[:, 0] == tt0).astype(jnp.float32)
    params = jnp.stack(
        [jnp.full((B,), delta), jnp.full((B,), tt0), ext] + [jnp.zeros((B,))] * 5,
        axis=1,
    ).astype(jnp.float32)

    ytab, gidx = _make_table_and_idx(x, t, W, b, params)
    out = _sc_gather(ytab.reshape(B * NPAD, D), gidx.reshape(NROWS))
    return out.reshape(B, T, T, 1, D)


# speedup vs baseline: 1.0069x; 1.0069x over previous
import functools

import jax
import jax.numpy as jnp
from jax import lax
from jax.experimental import pallas as pl
from jax.experimental.pallas import tpu as pltpu
from jax.experimental.pallas import tpu_sc as plsc

B, T, D = 4, 256, 128
NPAD = 2568
NROWS = B * T * T
NC, NS = 2, 16
NW = NC * NS
ROWS_PER_W = NROWS // NW
CHUNK = 128
SUP = 128
GPB = SUP // CHUNK
NSUP = ROWS_PER_W // SUP
NBUF = 6
LOOPED = 60


def _tc_body(t_ref, tT_ref, x_ref, W_ref, b_ref, p_ref, ytab_ref, gidx_ref):
    bi = pl.program_id(0)
    delta = p_ref[0, 0, 0]
    tt0 = p_ref[0, 0, 1]
    ext = p_ref[0, 0, 2]

    tr = t_ref[0]
    s = lax.broadcasted_iota(jnp.int32, (NPAD, 1), 0).astype(jnp.float32) * delta

    cmp = (tr <= s).astype(jnp.int32)
    cnt = jnp.sum(cmp, axis=1, keepdims=True)
    cp = jnp.clip(cnt, 1, T - 1)
    kk = lax.broadcasted_iota(jnp.int32, (NPAD, T), 1)
    m1 = (kk == cp - 1).astype(jnp.float32)
    m2 = (kk == cp).astype(jnp.float32)
    t_prev = jnp.sum(m1 * tr, axis=1, keepdims=True)
    t_next = jnp.sum(m2 * tr, axis=1, keepdims=True)
    u = (s - t_prev) / (t_next - t_prev)
    u = jnp.minimum(u, 1.0)
    keep_neg = jnp.logical_and(ext > 0.0, s < tt0)
    u = jnp.where(keep_neg, u, jnp.maximum(u, 0.0))

    A = m1 + u * (m2 - m1)
    xW = lax.dot_general(x_ref[0], W_ref[...],
                         (((1,), (1,)), ((), ())),
                         preferred_element_type=jnp.float32)
    y = lax.dot_general(A, xW, (((1,), (0,)), ((), ())),
                        preferred_element_type=jnp.float32) + b_ref[0:1, :]
    ytab_ref[0] = y

    tcol = tT_ref[0]
    it = tr + (tcol - tr) * jnp.float32(0.5)
    disc = (it / jnp.float32(0.1)).astype(jnp.int32)
    gidx_ref[0] = bi * NPAD + disc


def _make_table_and_idx(x, t, W, b, params):
    t3 = t.reshape(B, 1, T)
    tT = t.reshape(B, T, 1)
    b2 = b.reshape(1, D)
    p3 = params.reshape(B, 1, 8)
    return pl.pallas_call(
        _tc_body,
        grid=(B,),
        in_specs=[
            pl.BlockSpec((1, 1, T), lambda i: (i, 0, 0)),
            pl.BlockSpec((1, T, 1), lambda i: (i, 0, 0)),
            pl.BlockSpec((1, T, D), lambda i: (i, 0, 0)),
            pl.BlockSpec((D, D), lambda i: (0, 0)),
            pl.BlockSpec((1, D), lambda i: (0, 0)),
            pl.BlockSpec((1, 1, 8), lambda i: (i, 0, 0)),
        ],
        out_specs=[
            pl.BlockSpec((1, NPAD, D), lambda i: (i, 0, 0)),
            pl.BlockSpec((1, T, T), lambda i: (i, 0, 0)),
        ],
        out_shape=[
            jax.ShapeDtypeStruct((B, NPAD, D), jnp.float32),
            jax.ShapeDtypeStruct((B, T, T), jnp.int32),
        ],
    )(t3, tT, x, W, b2, p3)


@functools.partial(
    pl.kernel,
    mesh=plsc.VectorSubcoreMesh(core_axis_name="c", subcore_axis_name="s"),
    out_type=jax.ShapeDtypeStruct((NROWS, D), jnp.float32),
    scratch_types=[
        pltpu.VMEM((ROWS_PER_W,), jnp.int32),
    ] + [pltpu.VMEM((SUP, D), jnp.float32)] * NBUF
      + [pltpu.SemaphoreType.DMA] * (2 * NBUF),
)
def _sc_gather(ytab_hbm, gidx_hbm, out_hbm, idx_all, *bufsems):
    rows = bufsems[:NBUF]
    gsem = bufsems[NBUF:2 * NBUF]
    wsem = bufsems[2 * NBUF:]
    wid = lax.axis_index("s") * NC + lax.axis_index("c")
    wbase = wid * ROWS_PER_W

    pltpu.sync_copy(gidx_hbm.at[pl.ds(wbase, ROWS_PER_W)], idx_all)

    def fire_gathers(s, kb):
        for h in range(GPB):
            pltpu.async_copy(
                ytab_hbm.at[idx_all.at[pl.ds(s * SUP + h * CHUNK, CHUNK)]],
                rows[kb].at[pl.ds(h * CHUNK, CHUNK)],
                gsem[kb])

    def drain(sem, kb):
        pltpu.make_async_copy(ytab_hbm.at[pl.ds(0, SUP)], rows[kb], sem).wait()

    def fire_wb(s, kb):
        pltpu.async_copy(rows[kb],
                         out_hbm.at[pl.ds(wbase + s * SUP, SUP)],
                         wsem[kb])

    def step(s, k):
        drain(gsem[k], k)
        kn = (k + NBUF - 1) % NBUF

        @pl.when(s + NBUF - 1 < NSUP)
        def _fire_next():
            @pl.when(s >= 1)
            def _wait_buf():
                drain(wsem[kn], kn)
            fire_gathers(s + NBUF - 1, kn)

        fire_wb(s, k)

    for k in range(NBUF - 1):
        fire_gathers(k, k)

    def body(o, carry):
        for k in range(NBUF):
            step(o * NBUF + k, k)
        return carry

    lax.fori_loop(0, LOOPED // NBUF, body, 0)
    for s in range(LOOPED, NSUP):
        step(s, s % NBUF)
    for k in range(NBUF):
        drain(wsem[k], k)


def kernel(x, t, W, b):
    t = t.astype(jnp.float32)
    x = x.astype(jnp.float32)
    tmax = jnp.max(t)
    a8 = 8.0 * tmax
    a2 = 2.0 * tmax
    f8 = jnp.floor(a8)
    f2 = jnp.floor(a2)
    N = (f8 + f2 + jnp.floor((a8 - f8) + (a2 - f2))).astype(jnp.int32) + 6
    delta = (tmax + 5 * 0.1) / (N - 1).astype(jnp.float32)
    tt0 = jnp.min(t)
    ext = (t[:, 0] == tt0).astype(jnp.float32)
    params = jnp.stack(
        [jnp.full((B,), delta), jnp.full((B,), tt0), ext] + [jnp.zeros((B,))] * 5,
        axis=1,
    ).astype(jnp.float32)

    ytab, gidx = _make_table_and_idx(x, t, W, b, params)
    out = _sc_gather(ytab.reshape(B * NPAD, D), gidx.reshape(NROWS))
    return out.reshape(B, T, T, 1, D)


# speedup vs baseline: 1.4118x; 1.4020x over previous
import functools

import jax
import jax.numpy as jnp
from jax import lax
from jax.experimental import pallas as pl
from jax.experimental.pallas import tpu as pltpu
from jax.experimental.pallas import tpu_sc as plsc

B, T, D = 4, 256, 128
NPAD = 2568
NROWS = B * T * T
NC, NS = 2, 16
NW = NC * NS
ROWS_PER_W = NROWS // NW
CHUNK = 128
SUP = 128
GPB = SUP // CHUNK
NSUP = ROWS_PER_W // SUP
NBUF = 2
LOOPED = 64


def _tc_body(t_ref, tT_ref, x_ref, W_ref, b_ref, p_ref, ytab_ref, gidx_ref):
    bi = pl.program_id(0)
    delta = p_ref[0, 0, 0]
    tt0 = p_ref[0, 0, 1]
    ext = p_ref[0, 0, 2]

    tr = t_ref[0]
    s = lax.broadcasted_iota(jnp.int32, (NPAD, 1), 0).astype(jnp.float32) * delta

    cmp = (tr <= s).astype(jnp.int32)
    cnt = jnp.sum(cmp, axis=1, keepdims=True)
    cp = jnp.clip(cnt, 1, T - 1)
    kk = lax.broadcasted_iota(jnp.int32, (NPAD, T), 1)
    m1 = (kk == cp - 1).astype(jnp.float32)
    m2 = (kk == cp).astype(jnp.float32)
    t_prev = jnp.sum(m1 * tr, axis=1, keepdims=True)
    t_next = jnp.sum(m2 * tr, axis=1, keepdims=True)
    u = (s - t_prev) / (t_next - t_prev)
    u = jnp.minimum(u, 1.0)
    keep_neg = jnp.logical_and(ext > 0.0, s < tt0)
    u = jnp.where(keep_neg, u, jnp.maximum(u, 0.0))

    A = m1 + u * (m2 - m1)
    xW = lax.dot_general(x_ref[0], W_ref[...],
                         (((1,), (1,)), ((), ())),
                         preferred_element_type=jnp.float32)
    y = lax.dot_general(A, xW, (((1,), (0,)), ((), ())),
                        preferred_element_type=jnp.float32) + b_ref[0:1, :]
    ytab_ref[0] = y

    tcol = tT_ref[0]
    it = tr + (tcol - tr) * jnp.float32(0.5)
    disc = (it / jnp.float32(0.1)).astype(jnp.int32)
    gidx_ref[0] = bi * NPAD + disc


def _make_table_and_idx(x, t, W, b, params):
    t3 = t.reshape(B, 1, T)
    tT = t.reshape(B, T, 1)
    b2 = b.reshape(1, D)
    p3 = params.reshape(B, 1, 8)
    return pl.pallas_call(
        _tc_body,
        grid=(B,),
        in_specs=[
            pl.BlockSpec((1, 1, T), lambda i: (i, 0, 0)),
            pl.BlockSpec((1, T, 1), lambda i: (i, 0, 0)),
            pl.BlockSpec((1, T, D), lambda i: (i, 0, 0)),
            pl.BlockSpec((D, D), lambda i: (0, 0)),
            pl.BlockSpec((1, D), lambda i: (0, 0)),
            pl.BlockSpec((1, 1, 8), lambda i: (i, 0, 0)),
        ],
        out_specs=[
            pl.BlockSpec((1, NPAD, D), lambda i: (i, 0, 0)),
            pl.BlockSpec((1, T, T), lambda i: (i, 0, 0)),
        ],
        out_shape=[
            jax.ShapeDtypeStruct((B, NPAD, D), jnp.float32),
            jax.ShapeDtypeStruct((B, T, T), jnp.int32),
        ],
    )(t3, tT, x, W, b2, p3)


@functools.partial(
    pl.kernel,
    mesh=plsc.VectorSubcoreMesh(core_axis_name="c", subcore_axis_name="s"),
    out_type=jax.ShapeDtypeStruct((NROWS, D), jnp.float32),
    scratch_types=[
        pltpu.VMEM((ROWS_PER_W,), jnp.int32),
        pltpu.VMEM_SHARED((B * NPAD, D), jnp.float32),
    ] + [pltpu.VMEM((SUP, D), jnp.float32)] * NBUF
      + [pltpu.SemaphoreType.DMA] * (2 * NBUF),
)
def _sc_gather(ytab_hbm, gidx_hbm, out_hbm, idx_all, stab, *bufsems):
    rows = bufsems[:NBUF]
    gsem = bufsems[NBUF:2 * NBUF]
    wsem = bufsems[2 * NBUF:]
    sid = lax.axis_index("s")
    wid = sid * NC + lax.axis_index("c")
    wbase = wid * ROWS_PER_W

    pltpu.sync_copy(ytab_hbm.at[pl.ds(sid * 640, 640)],
                    stab.at[pl.ds(sid * 640, 640)])

    @pl.when(sid == NS - 1)
    def _tail():
        pltpu.sync_copy(ytab_hbm.at[pl.ds(NS * 640, B * NPAD - NS * 640)],
                        stab.at[pl.ds(NS * 640, B * NPAD - NS * 640)])
    pltpu.sync_copy(gidx_hbm.at[pl.ds(wbase, ROWS_PER_W)], idx_all)
    plsc.subcore_barrier()

    def fire_gathers(s, kb):
        for h in range(GPB):
            pltpu.async_copy(
                stab.at[idx_all.at[pl.ds(s * SUP + h * CHUNK, CHUNK)]],
                rows[kb].at[pl.ds(h * CHUNK, CHUNK)],
                gsem[kb])

    def drain(sem, kb):
        pltpu.make_async_copy(ytab_hbm.at[pl.ds(0, SUP)], rows[kb], sem).wait()

    def fire_wb(s, kb):
        pltpu.async_copy(rows[kb],
                         out_hbm.at[pl.ds(wbase + s * SUP, SUP)],
                         wsem[kb])

    def step(s, k):
        drain(gsem[k], k)
        kn = (k + NBUF - 1) % NBUF

        @pl.when(s + NBUF - 1 < NSUP)
        def _fire_next():
            @pl.when(s >= 1)
            def _wait_buf():
                drain(wsem[kn], kn)
            fire_gathers(s + NBUF - 1, kn)

        fire_wb(s, k)

    for k in range(NBUF - 1):
        fire_gathers(k, k)

    def body(o, carry):
        for k in range(NBUF):
            step(o * NBUF + k, k)
        return carry

    lax.fori_loop(0, LOOPED // NBUF, body, 0)
    for s in range(LOOPED, NSUP):
        step(s, s % NBUF)
    for k in range(NBUF):
        drain(wsem[k], k)


def kernel(x, t, W, b):
    t = t.astype(jnp.float32)
    x = x.astype(jnp.float32)
    tmax = jnp.max(t)
    a8 = 8.0 * tmax
    a2 = 2.0 * tmax
    f8 = jnp.floor(a8)
    f2 = jnp.floor(a2)
    N = (f8 + f2 + jnp.floor((a8 - f8) + (a2 - f2))).astype(jnp.int32) + 6
    delta = (tmax + 5 * 0.1) / (N - 1).astype(jnp.float32)
    tt0 = jnp.min(t)
    ext = (t[:, 0] == tt0).astype(jnp.float32)
    params = jnp.stack(
        [jnp.full((B,), delta), jnp.full((B,), tt0), ext] + [jnp.zeros((B,))] * 5,
        axis=1,
    ).astype(jnp.float32)

    ytab, gidx = _make_table_and_idx(x, t, W, b, params)
    out = _sc_gather(ytab.reshape(B * NPAD, D), gidx.reshape(NROWS))
    return out.reshape(B, T, T, 1, D)


# speedup vs baseline: 1.4309x; 1.0136x over previous
import functools

import jax
import jax.numpy as jnp
from jax import lax
from jax.experimental import pallas as pl
from jax.experimental.pallas import tpu as pltpu
from jax.experimental.pallas import tpu_sc as plsc

B, T, D = 4, 256, 128
NPAD = 2568
NROWS = B * T * T
NC, NS = 2, 16
NW = NC * NS
ROWS_PER_W = NROWS // NW
CHUNK = 128
SUP = 128
GPB = SUP // CHUNK
NSUP = ROWS_PER_W // SUP
NBUF = 2
LOOPED = 64
TTILE = 856


def _tc_body(t_ref, tT_ref, x_ref, W_ref, b_ref, p_ref, ytab_ref, gidx_ref):
    bi = pl.program_id(0)
    delta = p_ref[0, 0, 0]
    tt0 = p_ref[0, 0, 1]
    ext = p_ref[0, 0, 2]

    tr = t_ref[0]
    xW = lax.dot_general(x_ref[0], W_ref[...],
                         (((1,), (1,)), ((), ())),
                         preferred_element_type=jnp.float32)
    dmax = (tr[0, T - 1] / jnp.float32(0.1)).astype(jnp.int32)

    for tile in range(NPAD // TTILE):
        base = tile * TTILE

        @pl.when(jnp.int32(base) <= dmax)
        def _tile():
            s = (lax.broadcasted_iota(jnp.int32, (TTILE, 1), 0)
                 + base).astype(jnp.float32) * delta
            cmp = (tr <= s).astype(jnp.int32)
            cnt = jnp.sum(cmp, axis=1, keepdims=True)
            cp = jnp.clip(cnt, 1, T - 1)
            kk = lax.broadcasted_iota(jnp.int32, (TTILE, T), 1)
            m1 = (kk == cp - 1).astype(jnp.float32)
            m2 = (kk == cp).astype(jnp.float32)
            t_prev = jnp.sum(m1 * tr, axis=1, keepdims=True)
            t_next = jnp.sum(m2 * tr, axis=1, keepdims=True)
            u = (s - t_prev) / (t_next - t_prev)
            u = jnp.minimum(u, 1.0)
            keep_neg = jnp.logical_and(ext > 0.0, s < tt0)
            u = jnp.where(keep_neg, u, jnp.maximum(u, 0.0))

            A = m1 + u * (m2 - m1)
            y = lax.dot_general(A, xW, (((1,), (0,)), ((), ())),
                                preferred_element_type=jnp.float32) + b_ref[0:1, :]
            ytab_ref[0, base:base + TTILE, :] = y

    tcol = tT_ref[0]
    it = tr + (tcol - tr) * jnp.float32(0.5)
    disc = (it / jnp.float32(0.1)).astype(jnp.int32)
    gidx_ref[0] = bi * NPAD + disc


def _make_table_and_idx(x, t, W, b, params):
    t3 = t.reshape(B, 1, T)
    tT = t.reshape(B, T, 1)
    b2 = b.reshape(1, D)
    p3 = params.reshape(B, 1, 8)
    return pl.pallas_call(
        _tc_body,
        grid=(B,),
        in_specs=[
            pl.BlockSpec((1, 1, T), lambda i: (i, 0, 0)),
            pl.BlockSpec((1, T, 1), lambda i: (i, 0, 0)),
            pl.BlockSpec((1, T, D), lambda i: (i, 0, 0)),
            pl.BlockSpec((D, D), lambda i: (0, 0)),
            pl.BlockSpec((1, D), lambda i: (0, 0)),
            pl.BlockSpec((1, 1, 8), lambda i: (i, 0, 0)),
        ],
        out_specs=[
            pl.BlockSpec((1, NPAD, D), lambda i: (i, 0, 0)),
            pl.BlockSpec((1, T, T), lambda i: (i, 0, 0)),
        ],
        out_shape=[
            jax.ShapeDtypeStruct((B, NPAD, D), jnp.float32),
            jax.ShapeDtypeStruct((B, T, T), jnp.int32),
        ],
    )(t3, tT, x, W, b2, p3)


@functools.partial(
    pl.kernel,
    mesh=plsc.VectorSubcoreMesh(core_axis_name="c", subcore_axis_name="s"),
    out_type=jax.ShapeDtypeStruct((NROWS, D), jnp.float32),
    scratch_types=[
        pltpu.VMEM((ROWS_PER_W,), jnp.int32),
        pltpu.VMEM_SHARED((B * NPAD, D), jnp.float32),
    ] + [pltpu.VMEM((SUP, D), jnp.float32)] * NBUF
      + [pltpu.SemaphoreType.DMA] * (2 * NBUF),
)
def _sc_gather(ytab_hbm, gidx_hbm, out_hbm, idx_all, stab, *bufsems):
    rows = bufsems[:NBUF]
    gsem = bufsems[NBUF:2 * NBUF]
    wsem = bufsems[2 * NBUF:]
    sid = lax.axis_index("s")
    wid = sid * NC + lax.axis_index("c")
    wbase = wid * ROWS_PER_W

    pltpu.sync_copy(ytab_hbm.at[pl.ds(sid * 640, 640)],
                    stab.at[pl.ds(sid * 640, 640)])

    @pl.when(sid == NS - 1)
    def _tail():
        pltpu.sync_copy(ytab_hbm.at[pl.ds(NS * 640, B * NPAD - NS * 640)],
                        stab.at[pl.ds(NS * 640, B * NPAD - NS * 640)])
    pltpu.sync_copy(gidx_hbm.at[pl.ds(wbase, ROWS_PER_W)], idx_all)
    plsc.subcore_barrier()

    def fire_gathers(s, kb):
        for h in range(GPB):
            pltpu.async_copy(
                stab.at[idx_all.at[pl.ds(s * SUP + h * CHUNK, CHUNK)]],
                rows[kb].at[pl.ds(h * CHUNK, CHUNK)],
                gsem[kb])

    def drain(sem, kb):
        pltpu.make_async_copy(ytab_hbm.at[pl.ds(0, SUP)], rows[kb], sem).wait()

    def fire_wb(s, kb):
        pltpu.async_copy(rows[kb],
                         out_hbm.at[pl.ds(wbase + s * SUP, SUP)],
                         wsem[kb])

    def step(s, k):
        drain(gsem[k], k)
        kn = (k + NBUF - 1) % NBUF

        @pl.when(s + NBUF - 1 < NSUP)
        def _fire_next():
            @pl.when(s >= 1)
            def _wait_buf():
                drain(wsem[kn], kn)
            fire_gathers(s + NBUF - 1, kn)

        fire_wb(s, k)

    for k in range(NBUF - 1):
        fire_gathers(k, k)

    def body(o, carry):
        for k in range(NBUF):
            step(o * NBUF + k, k)
        return carry

    lax.fori_loop(0, LOOPED // NBUF, body, 0)
    for s in range(LOOPED, NSUP):
        step(s, s % NBUF)
    for k in range(NBUF):
        drain(wsem[k], k)


def kernel(x, t, W, b):
    t = t.astype(jnp.float32)
    x = x.astype(jnp.float32)
    tmax = jnp.max(t)
    a8 = 8.0 * tmax
    a2 = 2.0 * tmax
    f8 = jnp.floor(a8)
    f2 = jnp.floor(a2)
    N = (f8 + f2 + jnp.floor((a8 - f8) + (a2 - f2))).astype(jnp.int32) + 6
    delta = (tmax + 5 * 0.1) / (N - 1).astype(jnp.float32)
    tt0 = jnp.min(t)
    ext = (t[:, 0] == tt0).astype(jnp.float32)
    params = jnp.stack(
        [jnp.full((B,), delta), jnp.full((B,), tt0), ext] + [jnp.zeros((B,))] * 5,
        axis=1,
    ).astype(jnp.float32)

    ytab, gidx = _make_table_and_idx(x, t, W, b, params)
    out = _sc_gather(ytab.reshape(B * NPAD, D), gidx.reshape(NROWS))
    return out.reshape(B, T, T, 1, D)


# speedup vs baseline: 1.4852x; 1.0379x over previous
import functools

import jax
import jax.numpy as jnp
from jax import lax
from jax.experimental import pallas as pl
from jax.experimental.pallas import tpu as pltpu
from jax.experimental.pallas import tpu_sc as plsc

B, T, D = 4, 256, 128
NPAD = 2568
NROWS = B * T * T
NC, NS = 2, 16
NW = NC * NS
ROWS_PER_W = NROWS // NW
CHUNK = 128
SUP = 128
GPB = SUP // CHUNK
NSUP = ROWS_PER_W // SUP
NBUF = 2
LOOPED = 64
TTILE = 856


def _tc_body(t_ref, tT_ref, x_ref, W_ref, b_ref, p_ref, ytab_ref, gidx_ref):
    bi = pl.program_id(0)
    delta = p_ref[0, 0, 0]
    tt0 = p_ref[0, 0, 1]
    ext = p_ref[0, 0, 2]

    tr = t_ref[0]
    xW = lax.dot_general(x_ref[0], W_ref[...],
                         (((1,), (1,)), ((), ())),
                         preferred_element_type=jnp.float32)
    tshift = jnp.concatenate([tr[:, 1:], tr[:, T - 1:] + 1.0], axis=1)
    invdt = 1.0 / (tshift - tr)
    dxW = jnp.concatenate([xW[1:] - xW[:-1], jnp.zeros((1, D), jnp.float32)],
                          axis=0)
    y0 = xW[0:1, :] + b_ref[0:1, :]
    lane0 = lax.broadcasted_iota(jnp.int32, (1, T), 1) == 0

    dmax = (tr[0, T - 1] / jnp.float32(0.1)).astype(jnp.int32)

    for tile in range(NPAD // TTILE):
        base = tile * TTILE

        @pl.when(jnp.int32(base) <= dmax)
        def _tile():
            s = (lax.broadcasted_iota(jnp.int32, (TTILE, 1), 0)
                 + base).astype(jnp.float32) * delta
            g = jnp.minimum((s - tr) * invdt, 1.0)
            keep_neg = jnp.logical_and(ext > 0.0, s < tt0)
            lb = jnp.where(jnp.logical_and(keep_neg, lane0),
                           jnp.float32(-3.0e38), jnp.float32(0.0))
            g = jnp.maximum(g, lb)
            y = lax.dot_general(g, dxW, (((1,), (0,)), ((), ())),
                                preferred_element_type=jnp.float32) + y0
            ytab_ref[0, base:base + TTILE, :] = y

    tcol = tT_ref[0]
    it = tr + (tcol - tr) * jnp.float32(0.5)
    disc = (it / jnp.float32(0.1)).astype(jnp.int32)
    gidx_ref[0] = bi * NPAD + disc


def _make_table_and_idx(x, t, W, b, params):
    t3 = t.reshape(B, 1, T)
    tT = t.reshape(B, T, 1)
    b2 = b.reshape(1, D)
    p3 = params.reshape(B, 1, 8)
    return pl.pallas_call(
        _tc_body,
        grid=(B,),
        in_specs=[
            pl.BlockSpec((1, 1, T), lambda i: (i, 0, 0)),
            pl.BlockSpec((1, T, 1), lambda i: (i, 0, 0)),
            pl.BlockSpec((1, T, D), lambda i: (i, 0, 0)),
            pl.BlockSpec((D, D), lambda i: (0, 0)),
            pl.BlockSpec((1, D), lambda i: (0, 0)),
            pl.BlockSpec((1, 1, 8), lambda i: (i, 0, 0)),
        ],
        out_specs=[
            pl.BlockSpec((1, NPAD, D), lambda i: (i, 0, 0)),
            pl.BlockSpec((1, T, T), lambda i: (i, 0, 0)),
        ],
        out_shape=[
            jax.ShapeDtypeStruct((B, NPAD, D), jnp.float32),
            jax.ShapeDtypeStruct((B, T, T), jnp.int32),
        ],
    )(t3, tT, x, W, b2, p3)


@functools.partial(
    pl.kernel,
    mesh=plsc.VectorSubcoreMesh(core_axis_name="c", subcore_axis_name="s"),
    out_type=jax.ShapeDtypeStruct((NROWS, D), jnp.float32),
    scratch_types=[
        pltpu.VMEM((ROWS_PER_W,), jnp.int32),
        pltpu.VMEM_SHARED((B * NPAD, D), jnp.float32),
    ] + [pltpu.VMEM((SUP, D), jnp.float32)] * NBUF
      + [pltpu.SemaphoreType.DMA] * (2 * NBUF),
)
def _sc_gather(ytab_hbm, gidx_hbm, out_hbm, idx_all, stab, *bufsems):
    rows = bufsems[:NBUF]
    gsem = bufsems[NBUF:2 * NBUF]
    wsem = bufsems[2 * NBUF:]
    sid = lax.axis_index("s")
    wid = sid * NC + lax.axis_index("c")
    wbase = wid * ROWS_PER_W

    pltpu.sync_copy(ytab_hbm.at[pl.ds(sid * 640, 640)],
                    stab.at[pl.ds(sid * 640, 640)])

    @pl.when(sid == NS - 1)
    def _tail():
        pltpu.sync_copy(ytab_hbm.at[pl.ds(NS * 640, B * NPAD - NS * 640)],
                        stab.at[pl.ds(NS * 640, B * NPAD - NS * 640)])
    pltpu.sync_copy(gidx_hbm.at[pl.ds(wbase, ROWS_PER_W)], idx_all)
    plsc.subcore_barrier()

    def fire_gathers(s, kb):
        for h in range(GPB):
            pltpu.async_copy(
                stab.at[idx_all.at[pl.ds(s * SUP + h * CHUNK, CHUNK)]],
                rows[kb].at[pl.ds(h * CHUNK, CHUNK)],
                gsem[kb])

    def drain(sem, kb):
        pltpu.make_async_copy(ytab_hbm.at[pl.ds(0, SUP)], rows[kb], sem).wait()

    def fire_wb(s, kb):
        pltpu.async_copy(rows[kb],
                         out_hbm.at[pl.ds(wbase + s * SUP, SUP)],
                         wsem[kb])

    def step(s, k):
        drain(gsem[k], k)
        kn = (k + NBUF - 1) % NBUF

        @pl.when(s + NBUF - 1 < NSUP)
        def _fire_next():
            @pl.when(s >= 1)
            def _wait_buf():
                drain(wsem[kn], kn)
            fire_gathers(s + NBUF - 1, kn)

        fire_wb(s, k)

    for k in range(NBUF - 1):
        fire_gathers(k, k)

    def body(o, carry):
        for k in range(NBUF):
            step(o * NBUF + k, k)
        return carry

    lax.fori_loop(0, LOOPED // NBUF, body, 0)
    for s in range(LOOPED, NSUP):
        step(s, s % NBUF)
    for k in range(NBUF):
        drain(wsem[k], k)


def kernel(x, t, W, b):
    t = t.astype(jnp.float32)
    x = x.astype(jnp.float32)
    tmax = jnp.max(t)
    a8 = 8.0 * tmax
    a2 = 2.0 * tmax
    f8 = jnp.floor(a8)
    f2 = jnp.floor(a2)
    N = (f8 + f2 + jnp.floor((a8 - f8) + (a2 - f2))).astype(jnp.int32) + 6
    delta = (tmax + 5 * 0.1) / (N - 1).astype(jnp.float32)
    tt0 = jnp.min(t)
    ext = (t[:, 0] == tt0).astype(jnp.float32)
    params = jnp.stack(
        [jnp.full((B,), delta), jnp.full((B,), tt0), ext] + [jnp.zeros((B,))] * 5,
        axis=1,
    ).astype(jnp.float32)

    ytab, gidx = _make_table_and_idx(x, t, W, b, params)
    out = _sc_gather(ytab.reshape(B * NPAD, D), gidx.reshape(NROWS))
    return out.reshape(B, T, T, 1, D)
